# baseline (speedup 1.0000x reference)
import jax
import jax.numpy as jnp
from jax import lax
from jax.experimental import pallas as pl
from jax.experimental.pallas import tpu as pltpu
from jax.experimental.pallas import tpu_sc as plsc

N_NODES = 10000
NP = 10240
D = 256
HF = 128
E = 160000
N_TILES = 16
ROWS_PER_TILE = NP // N_TILES
RB = 1024
GRID = NP // RB

CONV_CH = 64
CONV_GRP = 32
CONV_NGRP = 5
CONV_K = 4
EPAD_TILE = CONV_CH * CONV_GRP * CONV_NGRP
EPAD = N_TILES * EPAD_TILE

_F32 = jnp.float32
_HIGH = lax.Precision.HIGHEST


DEG_CH = 40
DEG_NCH = 125
DEG_K = 5
DEG_NIT = DEG_NCH // DEG_K


def _deg_body(d1, d2, d3, o, acc, idx_v, one1_v, one2_v, one3_v, stage_v,
              s0, s1, s2, s3, s4):
    sems = (s0, s1, s2, s3, s4)
    c = lax.axis_index("c")
    s = lax.axis_index("s")
    w = s * 2 + c
    z = jnp.zeros((16,), _F32)
    one = jnp.ones((16,), _F32)

    def fill_stage(k, carry):
        def fcol(j, inner):
            stage_v[k, pl.ds(j * 16, 16)] = z
            return inner
        return lax.fori_loop(0, 3, fcol, carry)
    lax.fori_loop(0, 80, fill_stage, 0)

    for kset, buf in enumerate((one1_v, one2_v, one3_v)):
        def fill_ones(k, carry):
            for j in range(3):
                buf[k, pl.ds(j * 16, 16)] = one if j == kset else z
            return carry
        lax.fori_loop(0, DEG_CH, fill_ones, 0)

    for k in range(ROWS_PER_TILE // 80):
        r0 = pl.multiple_of(s * ROWS_PER_TILE + k * 80, 8)
        pltpu.sync_copy(stage_v, acc.at[pl.ds(r0, 80)])
    plsc.subcore_barrier()

    for d, buf in ((d1, one1_v), (d2, one2_v), (d3, one3_v)):
        pltpu.sync_copy(d.at[w], idx_v)

        def scat(i, carry):
            cps = []
            for j in range(DEG_K):
                g = i * DEG_K + j
                cps.append(pltpu.async_copy(
                    buf, acc.at[idx_v.at[g]], sems[j], add=True))
            for cp in cps:
                cp.wait()
            return carry
        lax.fori_loop(0, DEG_NIT, scat, 0)

    plsc.subcore_barrier()
    for k in range(ROWS_PER_TILE // 80):
        r0 = pl.multiple_of(s * ROWS_PER_TILE + k * 80, 8)
        pltpu.sync_copy(acc.at[pl.ds(r0, 80)], stage_v)
        pltpu.sync_copy(stage_v, o.at[c, pl.ds(r0, 80)])


_deg_call = pl.kernel(
    _deg_body,
    out_type=jax.ShapeDtypeStruct((2, NP, 48), _F32),
    mesh=plsc.VectorSubcoreMesh(core_axis_name="c", subcore_axis_name="s"),
    scratch_types=[
        pltpu.VMEM_SHARED((NP, 48), _F32),
        pltpu.VMEM((DEG_NCH, DEG_CH), jnp.int32),
        pltpu.VMEM((DEG_CH, 48), _F32),
        pltpu.VMEM((DEG_CH, 48), _F32),
        pltpu.VMEM((DEG_CH, 48), _F32),
        pltpu.VMEM((80, 48), _F32),
        pltpu.SemaphoreType.DMA,
        pltpu.SemaphoreType.DMA,
        pltpu.SemaphoreType.DMA,
        pltpu.SemaphoreType.DMA,
        pltpu.SemaphoreType.DMA,
    ],
)


def _conv_body(tbl, src_hbm, dst_hbm, out, acc, sidx, didx, rows,
               g0, g1, g2, g3, t0, t1, t2, t3):
    gsem = (g0, g1, g2, g3)
    ssem = (t0, t1, t2, t3)
    c = lax.axis_index("c")
    s = lax.axis_index("s")
    z = jnp.zeros((16,), _F32)

    def zrow(r, carry):
        def zcol(j, inner):
            rows[0, r, pl.ds(j * 16, 16)] = z
            return inner
        return lax.fori_loop(0, HF // 16, zcol, carry)
    lax.fori_loop(0, CONV_CH, zrow, 0)
    for k in range(ROWS_PER_TILE // CONV_CH):
        r0 = pl.multiple_of(s * ROWS_PER_TILE + k * CONV_CH, 8)
        pltpu.sync_copy(rows.at[0], acc.at[pl.ds(r0, CONV_CH)])
    plsc.subcore_barrier()

    def gather_desc(q, j):
        return pltpu.make_async_copy(tbl.at[sidx.at[q]], rows.at[j], gsem[j])

    def scatter_desc(q, j):
        return pltpu.make_async_copy(rows.at[j], acc.at[didx.at[q]], ssem[j])

    def group(gi, carry):
        pltpu.sync_copy(src_hbm.at[c * N_TILES + s, gi], sidx)
        pltpu.sync_copy(dst_hbm.at[s, gi], didx)

        for j in range(CONV_K):
            gather_desc(j, j).start()

        def steady(it, carry2):
            for j in range(CONV_K):
                q = it * CONV_K + j
                gather_desc(q, j).wait()
                scatter_desc(q, j).start(add=True)
            for j in range(CONV_K):
                q = it * CONV_K + j
                scatter_desc(q, j).wait()
                gather_desc((it + 1) * CONV_K + j, j).start()
            return carry2
        lax.fori_loop(0, CONV_GRP // CONV_K - 1, steady, 0)

        last = CONV_GRP - CONV_K
        for j in range(CONV_K):
            gather_desc(last + j, j).wait()
            scatter_desc(last + j, j).start(add=True)
        for j in range(CONV_K):
            scatter_desc(last + j, j).wait()
        return carry
    lax.fori_loop(0, CONV_NGRP, group, 0)

    plsc.subcore_barrier()
    for k in range(ROWS_PER_TILE // CONV_CH):
        r0 = pl.multiple_of(s * ROWS_PER_TILE + k * CONV_CH, 8)
        pltpu.sync_copy(acc.at[pl.ds(r0, CONV_CH)], rows.at[0])
        pltpu.sync_copy(rows.at[0], out.at[c, pl.ds(r0, CONV_CH)])


_conv_call = pl.kernel(
    _conv_body,
    out_type=jax.ShapeDtypeStruct((2, NP, HF), _F32),
    mesh=plsc.VectorSubcoreMesh(core_axis_name="c", subcore_axis_name="s"),
    scratch_types=[
        pltpu.VMEM_SHARED((NP, HF), _F32),
        pltpu.VMEM((CONV_GRP, CONV_CH), jnp.int32),
        pltpu.VMEM((CONV_GRP, CONV_CH), jnp.int32),
        pltpu.VMEM((CONV_K, CONV_CH, HF), _F32),
        pltpu.SemaphoreType.DMA,
        pltpu.SemaphoreType.DMA,
        pltpu.SemaphoreType.DMA,
        pltpu.SemaphoreType.DMA,
        pltpu.SemaphoreType.DMA,
        pltpu.SemaphoreType.DMA,
        pltpu.SemaphoreType.DMA,
        pltpu.SemaphoreType.DMA,
    ],
)


def _dinv_from(dp, kset):
    lo = 16 * kset
    deg = 1.0 + dp[0, :, lo:lo + 1] + dp[1, :, lo:lo + 1]
    return deg, lax.rsqrt(deg)


_DP_SPEC = pl.BlockSpec((2, RB, 48), lambda i: (0, i, 0))
_HS_SPEC = pl.BlockSpec((2, RB, HF), lambda i: (0, i, 0))
_ROW_SPEC = pl.BlockSpec((RB, D), lambda i: (i, 0))
_HS_SHAPE = jax.ShapeDtypeStruct((2, NP, HF), _F32)
_ROW_SHAPE = jax.ShapeDtypeStruct((NP, D), _F32)


def _tc1a_body(x_ref, w_ref, out_ref):
    out_ref[...] = jnp.dot(x_ref[...], w_ref[...], precision=_HIGH,
                           preferred_element_type=_F32)


_tc1a = pl.pallas_call(
    _tc1a_body,
    grid=(GRID,),
    in_specs=[
        pl.BlockSpec((RB, D), lambda i: (i, 0)),
        pl.BlockSpec((D, D), lambda i: (0, 0)),
    ],
    out_specs=_ROW_SPEC,
    out_shape=_ROW_SHAPE,
)


def _tc1b_body(hm_ref, b_ref, dp_ref, hs_ref, base_ref):
    Hm = hm_ref[...]
    deg, dinv = _dinv_from(dp_ref[...], 0)
    hs = Hm * dinv
    hs_ref[0] = hs[:, :HF]
    hs_ref[1] = hs[:, HF:]
    base_ref[...] = Hm * (1.0 / deg) + b_ref[...]


_tc1b = pl.pallas_call(
    _tc1b_body,
    grid=(GRID,),
    in_specs=[
        _ROW_SPEC,
        pl.BlockSpec((1, D), lambda i: (0, 0)),
        _DP_SPEC,
    ],
    out_specs=[_HS_SPEC, _ROW_SPEC],
    out_shape=[_HS_SHAPE, _ROW_SHAPE],
)


def _make_tcmid(kprev, kcur):
    def body(acc_ref, bin_ref, dp_ref, w_ref, b_ref, hs_ref, base_ref):
        dp = dp_ref[...]
        _, dinvp = _dinv_from(dp, kprev)
        a = acc_ref[...]
        bi = bin_ref[...]
        Xa = a[0] * dinvp + bi[:, :HF]
        Xb = a[1] * dinvp + bi[:, HF:]
        W = w_ref[...]
        Hm = (jnp.dot(Xa, W[:HF, :], precision=_HIGH,
                      preferred_element_type=_F32)
              + jnp.dot(Xb, W[HF:, :], precision=_HIGH,
                        preferred_element_type=_F32))
        deg, dinv = _dinv_from(dp, kcur)
        hs = Hm * dinv
        hs_ref[0] = hs[:, :HF]
        hs_ref[1] = hs[:, HF:]
        base_ref[...] = Hm * (1.0 / deg) + b_ref[...]

    return pl.pallas_call(
        body,
        grid=(GRID,),
        in_specs=[
            _HS_SPEC,
            _ROW_SPEC,
            _DP_SPEC,
            pl.BlockSpec((D, D), lambda i: (0, 0)),
            pl.BlockSpec((1, D), lambda i: (0, 0)),
        ],
        out_specs=[_HS_SPEC, _ROW_SPEC],
        out_shape=[_HS_SHAPE, _ROW_SHAPE],
    )


_tcmid01 = _make_tcmid(0, 1)
_tcmid12 = _make_tcmid(1, 2)


def _tcf_body(acc_ref, bin_ref, dp_ref, out_ref):
    _, dinvp = _dinv_from(dp_ref[...], 2)
    a = acc_ref[...]
    bi = bin_ref[...]
    Xa = a[0] * dinvp + bi[:, :HF]
    Xb = a[1] * dinvp + bi[:, HF:]
    out_ref[...] = jnp.concatenate([Xa, Xb], axis=1)


_tcf = pl.pallas_call(
    _tcf_body,
    grid=(GRID,),
    in_specs=[_HS_SPEC, _ROW_SPEC, _DP_SPEC],
    out_specs=pl.BlockSpec((RB, D), lambda i: (i, 0)),
    out_shape=jax.ShapeDtypeStruct((NP, D), _F32),
)


def kernel(embedding, down2up_path, same_level_edge_index, up2down_edge_index,
           W1, b1, W2, b2, W3, b3):
    edges = (down2up_path, same_level_edge_index, up2down_edge_index)
    srcs = [e[0].astype(jnp.int32) for e in edges]
    dsts = [e[1].astype(jnp.int32) for e in edges]

    npad = EPAD - E
    fake_src = (jnp.arange(npad, dtype=jnp.int32) * 37) % NP
    fake_dst = N_NODES + jnp.arange(npad, dtype=jnp.int32) % (NP - N_NODES)
    src_conv = []
    for sv in srcs:
        sp = jnp.concatenate([sv, fake_src])
        src_conv.append(jnp.stack([sp, sp + NP]).reshape(
            2 * N_TILES, CONV_NGRP, CONV_GRP, CONV_CH))
    dst_conv = [
        jnp.concatenate([dv, fake_dst]).reshape(
            N_TILES, CONV_NGRP, CONV_GRP, CONV_CH)
        for dv in dsts
    ]

    x = jnp.pad(embedding, ((0, NP - N_NODES), (0, 0)))
    b1r = b1.reshape(1, D)
    b2r = b2.reshape(1, D)
    b3r = b3.reshape(1, D)

    dst_deg = [d.reshape(2 * N_TILES, DEG_NCH, DEG_CH) for d in dsts]
    hm1 = _tc1a(x, W1)
    dp = _deg_call(*dst_deg)
    hs1, base1 = _tc1b(hm1, b1r, dp)
    acc1 = _conv_call(hs1.reshape(2 * NP, HF), src_conv[0], dst_conv[0])
    hs2, base2 = _tcmid01(acc1, base1, dp, W2, b2r)
    acc2 = _conv_call(hs2.reshape(2 * NP, HF), src_conv[1], dst_conv[1])
    hs3, base3 = _tcmid12(acc2, base2, dp, W3, b3r)
    acc3 = _conv_call(hs3.reshape(2 * NP, HF), src_conv[2], dst_conv[2])
    out = _tcf(acc3, base3, dp)
    return out[:N_NODES]


# speedup vs baseline: 1.0181x; 1.0181x over previous
import jax
import jax.numpy as jnp
from jax import lax
from jax.experimental import pallas as pl
from jax.experimental.pallas import tpu as pltpu
from jax.experimental.pallas import tpu_sc as plsc

N_NODES = 10000
NP = 10240
D = 256
HF = 128
E = 160000
N_TILES = 16
ROWS_PER_TILE = NP // N_TILES
RB = 1024
GRID = NP // RB

CONV_CH = 64
CONV_GRP = 40
CONV_NGRP = 4
CONV_K = 4
EPAD_TILE = CONV_CH * CONV_GRP * CONV_NGRP
EPAD = N_TILES * EPAD_TILE

_F32 = jnp.float32
_HIGH = lax.Precision.HIGHEST


DEG_CH = 40
DEG_NCH = 125
DEG_K = 5
DEG_NIT = DEG_NCH // DEG_K


def _deg_body(d1, d2, d3, o, acc, idx_v, one1_v, one2_v, one3_v, stage_v,
              s0, s1, s2, s3, s4):
    sems = (s0, s1, s2, s3, s4)
    c = lax.axis_index("c")
    s = lax.axis_index("s")
    w = s * 2 + c
    z = jnp.zeros((16,), _F32)
    one = jnp.ones((16,), _F32)

    def fill_stage(k, carry):
        def fcol(j, inner):
            stage_v[k, pl.ds(j * 16, 16)] = z
            return inner
        return lax.fori_loop(0, 3, fcol, carry)
    lax.fori_loop(0, 80, fill_stage, 0)

    for kset, buf in enumerate((one1_v, one2_v, one3_v)):
        def fill_ones(k, carry):
            for j in range(3):
                buf[k, pl.ds(j * 16, 16)] = one if j == kset else z
            return carry
        lax.fori_loop(0, DEG_CH, fill_ones, 0)

    for k in range(ROWS_PER_TILE // 80):
        r0 = pl.multiple_of(s * ROWS_PER_TILE + k * 80, 8)
        pltpu.sync_copy(stage_v, acc.at[pl.ds(r0, 80)])
    plsc.subcore_barrier()

    for d, buf in ((d1, one1_v), (d2, one2_v), (d3, one3_v)):
        pltpu.sync_copy(d.at[w], idx_v)

        def scat(i, carry):
            cps = []
            for j in range(DEG_K):
                g = i * DEG_K + j
                cps.append(pltpu.async_copy(
                    buf, acc.at[idx_v.at[g]], sems[j], add=True))
            for cp in cps:
                cp.wait()
            return carry
        lax.fori_loop(0, DEG_NIT, scat, 0)

    plsc.subcore_barrier()
    r0 = pl.multiple_of(s * ROWS_PER_TILE, 8)
    pltpu.sync_copy(acc.at[pl.ds(r0, ROWS_PER_TILE)],
                    o.at[c, pl.ds(r0, ROWS_PER_TILE)])


_deg_call = pl.kernel(
    _deg_body,
    out_type=jax.ShapeDtypeStruct((2, NP, 48), _F32),
    mesh=plsc.VectorSubcoreMesh(core_axis_name="c", subcore_axis_name="s"),
    scratch_types=[
        pltpu.VMEM_SHARED((NP, 48), _F32),
        pltpu.VMEM((DEG_NCH, DEG_CH), jnp.int32),
        pltpu.VMEM((DEG_CH, 48), _F32),
        pltpu.VMEM((DEG_CH, 48), _F32),
        pltpu.VMEM((DEG_CH, 48), _F32),
        pltpu.VMEM((80, 48), _F32),
        pltpu.SemaphoreType.DMA,
        pltpu.SemaphoreType.DMA,
        pltpu.SemaphoreType.DMA,
        pltpu.SemaphoreType.DMA,
        pltpu.SemaphoreType.DMA,
    ],
)


def _conv_body(tbl, src_hbm, dst_hbm, out, acc, sidx, didx, rows,
               g0, g1, g2, g3, t0, t1, t2, t3):
    gsem = (g0, g1, g2, g3)
    ssem = (t0, t1, t2, t3)
    c = lax.axis_index("c")
    s = lax.axis_index("s")
    z = jnp.zeros((16,), _F32)

    def zrow(r, carry):
        def zcol(j, inner):
            rows[0, r, pl.ds(j * 16, 16)] = z
            return inner
        return lax.fori_loop(0, HF // 16, zcol, carry)
    lax.fori_loop(0, CONV_CH, zrow, 0)
    for k in range(ROWS_PER_TILE // CONV_CH):
        r0 = pl.multiple_of(s * ROWS_PER_TILE + k * CONV_CH, 8)
        pltpu.sync_copy(rows.at[0], acc.at[pl.ds(r0, CONV_CH)])
    plsc.subcore_barrier()

    def gather_desc(q, j):
        return pltpu.make_async_copy(tbl.at[sidx.at[q]], rows.at[j], gsem[j])

    def scatter_desc(q, j):
        return pltpu.make_async_copy(rows.at[j], acc.at[didx.at[q]], ssem[j])

    def group(gi, carry):
        pltpu.sync_copy(src_hbm.at[c * N_TILES + s, gi], sidx)
        pltpu.sync_copy(dst_hbm.at[s, gi], didx)

        for j in range(CONV_K):
            gather_desc(j, j).start()

        def steady(it, carry2):
            for j in range(CONV_K):
                q = it * CONV_K + j
                gather_desc(q, j).wait()
                scatter_desc(q, j).start(add=True)
            for j in range(CONV_K):
                q = it * CONV_K + j
                scatter_desc(q, j).wait()
                gather_desc((it + 1) * CONV_K + j, j).start()
            return carry2
        lax.fori_loop(0, CONV_GRP // CONV_K - 1, steady, 0)

        last = CONV_GRP - CONV_K
        for j in range(CONV_K):
            gather_desc(last + j, j).wait()
            scatter_desc(last + j, j).start(add=True)
        for j in range(CONV_K):
            scatter_desc(last + j, j).wait()
        return carry
    lax.fori_loop(0, CONV_NGRP, group, 0)

    plsc.subcore_barrier()
    r0 = pl.multiple_of(s * ROWS_PER_TILE, 8)
    pltpu.sync_copy(acc.at[pl.ds(r0, ROWS_PER_TILE)],
                    out.at[c, pl.ds(r0, ROWS_PER_TILE)])


_conv_call = pl.kernel(
    _conv_body,
    out_type=jax.ShapeDtypeStruct((2, NP, HF), _F32),
    mesh=plsc.VectorSubcoreMesh(core_axis_name="c", subcore_axis_name="s"),
    scratch_types=[
        pltpu.VMEM_SHARED((NP, HF), _F32),
        pltpu.VMEM((CONV_GRP, CONV_CH), jnp.int32),
        pltpu.VMEM((CONV_GRP, CONV_CH), jnp.int32),
        pltpu.VMEM((CONV_K, CONV_CH, HF), _F32),
        pltpu.SemaphoreType.DMA,
        pltpu.SemaphoreType.DMA,
        pltpu.SemaphoreType.DMA,
        pltpu.SemaphoreType.DMA,
        pltpu.SemaphoreType.DMA,
        pltpu.SemaphoreType.DMA,
        pltpu.SemaphoreType.DMA,
        pltpu.SemaphoreType.DMA,
    ],
)


def _dinv_from(dp, kset):
    lo = 16 * kset
    deg = 1.0 + dp[0, :, lo:lo + 1] + dp[1, :, lo:lo + 1]
    return deg, lax.rsqrt(deg)


_DP_SPEC = pl.BlockSpec((2, RB, 48), lambda i: (0, i, 0))
_HS_SPEC = pl.BlockSpec((2, RB, HF), lambda i: (0, i, 0))
_ROW_SPEC = pl.BlockSpec((RB, D), lambda i: (i, 0))
_HS_SHAPE = jax.ShapeDtypeStruct((2, NP, HF), _F32)
_ROW_SHAPE = jax.ShapeDtypeStruct((NP, D), _F32)


def _tc1a_body(x_ref, w_ref, out_ref):
    out_ref[...] = jnp.dot(x_ref[...], w_ref[...], precision=_HIGH,
                           preferred_element_type=_F32)


_tc1a = pl.pallas_call(
    _tc1a_body,
    grid=(GRID,),
    in_specs=[
        pl.BlockSpec((RB, D), lambda i: (i, 0)),
        pl.BlockSpec((D, D), lambda i: (0, 0)),
    ],
    out_specs=_ROW_SPEC,
    out_shape=_ROW_SHAPE,
)


def _tc1b_body(hm_ref, b_ref, dp_ref, hs_ref, base_ref):
    Hm = hm_ref[...]
    deg, dinv = _dinv_from(dp_ref[...], 0)
    hs = Hm * dinv
    hs_ref[0] = hs[:, :HF]
    hs_ref[1] = hs[:, HF:]
    base_ref[...] = Hm * (1.0 / deg) + b_ref[...]


_tc1b = pl.pallas_call(
    _tc1b_body,
    grid=(GRID,),
    in_specs=[
        _ROW_SPEC,
        pl.BlockSpec((1, D), lambda i: (0, 0)),
        _DP_SPEC,
    ],
    out_specs=[_HS_SPEC, _ROW_SPEC],
    out_shape=[_HS_SHAPE, _ROW_SHAPE],
)


def _make_tcmid(kprev, kcur):
    def body(acc_ref, bin_ref, dp_ref, w_ref, b_ref, hs_ref, base_ref):
        dp = dp_ref[...]
        _, dinvp = _dinv_from(dp, kprev)
        a = acc_ref[...]
        bi = bin_ref[...]
        Xa = a[0] * dinvp + bi[:, :HF]
        Xb = a[1] * dinvp + bi[:, HF:]
        W = w_ref[...]
        Hm = (jnp.dot(Xa, W[:HF, :], precision=_HIGH,
                      preferred_element_type=_F32)
              + jnp.dot(Xb, W[HF:, :], precision=_HIGH,
                        preferred_element_type=_F32))
        deg, dinv = _dinv_from(dp, kcur)
        hs = Hm * dinv
        hs_ref[0] = hs[:, :HF]
        hs_ref[1] = hs[:, HF:]
        base_ref[...] = Hm * (1.0 / deg) + b_ref[...]

    return pl.pallas_call(
        body,
        grid=(GRID,),
        in_specs=[
            _HS_SPEC,
            _ROW_SPEC,
            _DP_SPEC,
            pl.BlockSpec((D, D), lambda i: (0, 0)),
            pl.BlockSpec((1, D), lambda i: (0, 0)),
        ],
        out_specs=[_HS_SPEC, _ROW_SPEC],
        out_shape=[_HS_SHAPE, _ROW_SHAPE],
    )


_tcmid01 = _make_tcmid(0, 1)
_tcmid12 = _make_tcmid(1, 2)


def _tcf_body(acc_ref, bin_ref, dp_ref, out_ref):
    _, dinvp = _dinv_from(dp_ref[...], 2)
    a = acc_ref[...]
    bi = bin_ref[...]
    Xa = a[0] * dinvp + bi[:, :HF]
    Xb = a[1] * dinvp + bi[:, HF:]
    out_ref[...] = jnp.concatenate([Xa, Xb], axis=1)


_tcf = pl.pallas_call(
    _tcf_body,
    grid=(GRID,),
    in_specs=[_HS_SPEC, _ROW_SPEC, _DP_SPEC],
    out_specs=pl.BlockSpec((RB, D), lambda i: (i, 0)),
    out_shape=jax.ShapeDtypeStruct((NP, D), _F32),
)


def kernel(embedding, down2up_path, same_level_edge_index, up2down_edge_index,
           W1, b1, W2, b2, W3, b3):
    edges = (down2up_path, same_level_edge_index, up2down_edge_index)
    srcs = [e[0].astype(jnp.int32) for e in edges]
    dsts = [e[1].astype(jnp.int32) for e in edges]

    npad = EPAD - E
    fake_src = (jnp.arange(npad, dtype=jnp.int32) * 37) % NP
    fake_dst = N_NODES + jnp.arange(npad, dtype=jnp.int32) % (NP - N_NODES)
    src_conv = []
    for sv in srcs:
        sp = jnp.concatenate([sv, fake_src])
        src_conv.append(jnp.stack([sp, sp + NP]).reshape(
            2 * N_TILES, CONV_NGRP, CONV_GRP, CONV_CH))
    dst_conv = [
        jnp.concatenate([dv, fake_dst]).reshape(
            N_TILES, CONV_NGRP, CONV_GRP, CONV_CH)
        for dv in dsts
    ]

    x = jnp.pad(embedding, ((0, NP - N_NODES), (0, 0)))
    b1r = b1.reshape(1, D)
    b2r = b2.reshape(1, D)
    b3r = b3.reshape(1, D)

    dst_deg = [d.reshape(2 * N_TILES, DEG_NCH, DEG_CH) for d in dsts]
    hm1 = _tc1a(x, W1)
    dp = _deg_call(*dst_deg)
    hs1, base1 = _tc1b(hm1, b1r, dp)
    acc1 = _conv_call(hs1.reshape(2 * NP, HF), src_conv[0], dst_conv[0])
    hs2, base2 = _tcmid01(acc1, base1, dp, W2, b2r)
    acc2 = _conv_call(hs2.reshape(2 * NP, HF), src_conv[1], dst_conv[1])
    hs3, base3 = _tcmid12(acc2, base2, dp, W3, b3r)
    acc3 = _conv_call(hs3.reshape(2 * NP, HF), src_conv[2], dst_conv[2])
    out = _tcf(acc3, base3, dp)
    return out[:N_NODES]


# speedup vs baseline: 1.0402x; 1.0217x over previous
import jax
import jax.numpy as jnp
from jax import lax
from jax.experimental import pallas as pl
from jax.experimental.pallas import tpu as pltpu
from jax.experimental.pallas import tpu_sc as plsc

N_NODES = 10000
NP = 10240
D = 256
HF = 128
E = 160000
N_TILES = 16
ROWS_PER_TILE = NP // N_TILES
RB = 1024
GRID = NP // RB

CONV_CH = 64
CONV_GRP = 40
CONV_NGRP = 4
CONV_K = 4
EPAD_TILE = CONV_CH * CONV_GRP * CONV_NGRP
EPAD = N_TILES * EPAD_TILE

_F32 = jnp.float32
_HIGH = lax.Precision.DEFAULT


DEG_CH = 40
DEG_NCH = 125
DEG_K = 5
DEG_NIT = DEG_NCH // DEG_K


def _deg_body(d1, d2, d3, o, acc, idx_v, one1_v, one2_v, one3_v, stage_v,
              s0, s1, s2, s3, s4):
    sems = (s0, s1, s2, s3, s4)
    c = lax.axis_index("c")
    s = lax.axis_index("s")
    w = s * 2 + c
    z = jnp.zeros((16,), _F32)
    one = jnp.ones((16,), _F32)

    def fill_stage(k, carry):
        def fcol(j, inner):
            stage_v[k, pl.ds(j * 16, 16)] = z
            return inner
        return lax.fori_loop(0, 3, fcol, carry)
    lax.fori_loop(0, 80, fill_stage, 0)

    for kset, buf in enumerate((one1_v, one2_v, one3_v)):
        def fill_ones(k, carry):
            for j in range(3):
                buf[k, pl.ds(j * 16, 16)] = one if j == kset else z
            return carry
        lax.fori_loop(0, DEG_CH, fill_ones, 0)

    for k in range(ROWS_PER_TILE // 80):
        r0 = pl.multiple_of(s * ROWS_PER_TILE + k * 80, 8)
        pltpu.sync_copy(stage_v, acc.at[pl.ds(r0, 80)])
    plsc.subcore_barrier()

    for d, buf in ((d1, one1_v), (d2, one2_v), (d3, one3_v)):
        pltpu.sync_copy(d.at[w], idx_v)

        def scat(i, carry):
            cps = []
            for j in range(DEG_K):
                g = i * DEG_K + j
                cps.append(pltpu.async_copy(
                    buf, acc.at[idx_v.at[g]], sems[j], add=True))
            for cp in cps:
                cp.wait()
            return carry
        lax.fori_loop(0, DEG_NIT, scat, 0)

    plsc.subcore_barrier()
    r0 = pl.multiple_of(s * ROWS_PER_TILE, 8)
    pltpu.sync_copy(acc.at[pl.ds(r0, ROWS_PER_TILE)],
                    o.at[c, pl.ds(r0, ROWS_PER_TILE)])


_deg_call = pl.kernel(
    _deg_body,
    out_type=jax.ShapeDtypeStruct((2, NP, 48), _F32),
    mesh=plsc.VectorSubcoreMesh(core_axis_name="c", subcore_axis_name="s"),
    scratch_types=[
        pltpu.VMEM_SHARED((NP, 48), _F32),
        pltpu.VMEM((DEG_NCH, DEG_CH), jnp.int32),
        pltpu.VMEM((DEG_CH, 48), _F32),
        pltpu.VMEM((DEG_CH, 48), _F32),
        pltpu.VMEM((DEG_CH, 48), _F32),
        pltpu.VMEM((80, 48), _F32),
        pltpu.SemaphoreType.DMA,
        pltpu.SemaphoreType.DMA,
        pltpu.SemaphoreType.DMA,
        pltpu.SemaphoreType.DMA,
        pltpu.SemaphoreType.DMA,
    ],
)


def _conv_body(tbl, src_hbm, dst_hbm, out, acc, sidx, didx, rows,
               g0, g1, g2, g3, t0, t1, t2, t3):
    gsem = (g0, g1, g2, g3)
    ssem = (t0, t1, t2, t3)
    c = lax.axis_index("c")
    s = lax.axis_index("s")
    z = jnp.zeros((16,), _F32)

    def zrow(r, carry):
        def zcol(j, inner):
            rows[0, r, pl.ds(j * 16, 16)] = z
            return inner
        return lax.fori_loop(0, HF // 16, zcol, carry)
    lax.fori_loop(0, CONV_CH, zrow, 0)
    for k in range(ROWS_PER_TILE // CONV_CH):
        r0 = pl.multiple_of(s * ROWS_PER_TILE + k * CONV_CH, 8)
        pltpu.sync_copy(rows.at[0], acc.at[pl.ds(r0, CONV_CH)])
    plsc.subcore_barrier()

    def gather_desc(q, j):
        return pltpu.make_async_copy(tbl.at[sidx.at[q]], rows.at[j], gsem[j])

    def scatter_desc(q, j):
        return pltpu.make_async_copy(rows.at[j], acc.at[didx.at[q]], ssem[j])

    def group(gi, carry):
        pltpu.sync_copy(src_hbm.at[c * N_TILES + s, gi], sidx)
        pltpu.sync_copy(dst_hbm.at[s, gi], didx)

        for j in range(CONV_K):
            gather_desc(j, j).start()

        def steady(it, carry2):
            for j in range(CONV_K):
                q = it * CONV_K + j
                gather_desc(q, j).wait()
                scatter_desc(q, j).start(add=True)
            for j in range(CONV_K):
                q = it * CONV_K + j
                scatter_desc(q, j).wait()
                gather_desc((it + 1) * CONV_K + j, j).start()
            return carry2
        lax.fori_loop(0, CONV_GRP // CONV_K - 1, steady, 0)

        last = CONV_GRP - CONV_K
        for j in range(CONV_K):
            gather_desc(last + j, j).wait()
            scatter_desc(last + j, j).start(add=True)
        for j in range(CONV_K):
            scatter_desc(last + j, j).wait()
        return carry
    lax.fori_loop(0, CONV_NGRP, group, 0)

    plsc.subcore_barrier()
    r0 = pl.multiple_of(s * ROWS_PER_TILE, 8)
    pltpu.sync_copy(acc.at[pl.ds(r0, ROWS_PER_TILE)],
                    out.at[c, pl.ds(r0, ROWS_PER_TILE)])


_conv_call = pl.kernel(
    _conv_body,
    out_type=jax.ShapeDtypeStruct((2, NP, HF), _F32),
    mesh=plsc.VectorSubcoreMesh(core_axis_name="c", subcore_axis_name="s"),
    scratch_types=[
        pltpu.VMEM_SHARED((NP, HF), _F32),
        pltpu.VMEM((CONV_GRP, CONV_CH), jnp.int32),
        pltpu.VMEM((CONV_GRP, CONV_CH), jnp.int32),
        pltpu.VMEM((CONV_K, CONV_CH, HF), _F32),
        pltpu.SemaphoreType.DMA,
        pltpu.SemaphoreType.DMA,
        pltpu.SemaphoreType.DMA,
        pltpu.SemaphoreType.DMA,
        pltpu.SemaphoreType.DMA,
        pltpu.SemaphoreType.DMA,
        pltpu.SemaphoreType.DMA,
        pltpu.SemaphoreType.DMA,
    ],
)


def _dinv_from(dp, kset):
    lo = 16 * kset
    deg = 1.0 + dp[0, :, lo:lo + 1] + dp[1, :, lo:lo + 1]
    return deg, lax.rsqrt(deg)


_DP_SPEC = pl.BlockSpec((2, RB, 48), lambda i: (0, i, 0))
_HS_SPEC = pl.BlockSpec((2, RB, HF), lambda i: (0, i, 0))
_ROW_SPEC = pl.BlockSpec((RB, D), lambda i: (i, 0))
_HS_SHAPE = jax.ShapeDtypeStruct((2, NP, HF), _F32)
_ROW_SHAPE = jax.ShapeDtypeStruct((NP, D), _F32)


def _tc1a_body(x_ref, w_ref, out_ref):
    out_ref[...] = jnp.dot(x_ref[...], w_ref[...], precision=_HIGH,
                           preferred_element_type=_F32)


_tc1a = pl.pallas_call(
    _tc1a_body,
    grid=(GRID,),
    in_specs=[
        pl.BlockSpec((RB, D), lambda i: (i, 0)),
        pl.BlockSpec((D, D), lambda i: (0, 0)),
    ],
    out_specs=_ROW_SPEC,
    out_shape=_ROW_SHAPE,
)


def _tc1b_body(hm_ref, b_ref, dp_ref, hs_ref, base_ref):
    Hm = hm_ref[...]
    deg, dinv = _dinv_from(dp_ref[...], 0)
    hs = Hm * dinv
    hs_ref[0] = hs[:, :HF]
    hs_ref[1] = hs[:, HF:]
    base_ref[...] = Hm * (1.0 / deg) + b_ref[...]


_tc1b = pl.pallas_call(
    _tc1b_body,
    grid=(GRID,),
    in_specs=[
        _ROW_SPEC,
        pl.BlockSpec((1, D), lambda i: (0, 0)),
        _DP_SPEC,
    ],
    out_specs=[_HS_SPEC, _ROW_SPEC],
    out_shape=[_HS_SHAPE, _ROW_SHAPE],
)


def _make_tcmid(kprev, kcur):
    def body(acc_ref, bin_ref, dp_ref, w_ref, b_ref, hs_ref, base_ref):
        dp = dp_ref[...]
        _, dinvp = _dinv_from(dp, kprev)
        a = acc_ref[...]
        bi = bin_ref[...]
        Xa = a[0] * dinvp + bi[:, :HF]
        Xb = a[1] * dinvp + bi[:, HF:]
        W = w_ref[...]
        Hm = (jnp.dot(Xa, W[:HF, :], precision=_HIGH,
                      preferred_element_type=_F32)
              + jnp.dot(Xb, W[HF:, :], precision=_HIGH,
                        preferred_element_type=_F32))
        deg, dinv = _dinv_from(dp, kcur)
        hs = Hm * dinv
        hs_ref[0] = hs[:, :HF]
        hs_ref[1] = hs[:, HF:]
        base_ref[...] = Hm * (1.0 / deg) + b_ref[...]

    return pl.pallas_call(
        body,
        grid=(GRID,),
        in_specs=[
            _HS_SPEC,
            _ROW_SPEC,
            _DP_SPEC,
            pl.BlockSpec((D, D), lambda i: (0, 0)),
            pl.BlockSpec((1, D), lambda i: (0, 0)),
        ],
        out_specs=[_HS_SPEC, _ROW_SPEC],
        out_shape=[_HS_SHAPE, _ROW_SHAPE],
    )


_tcmid01 = _make_tcmid(0, 1)
_tcmid12 = _make_tcmid(1, 2)


def _tcf_body(acc_ref, bin_ref, dp_ref, out_ref):
    _, dinvp = _dinv_from(dp_ref[...], 2)
    a = acc_ref[...]
    bi = bin_ref[...]
    Xa = a[0] * dinvp + bi[:, :HF]
    Xb = a[1] * dinvp + bi[:, HF:]
    out_ref[...] = jnp.concatenate([Xa, Xb], axis=1)


_tcf = pl.pallas_call(
    _tcf_body,
    grid=(GRID,),
    in_specs=[_HS_SPEC, _ROW_SPEC, _DP_SPEC],
    out_specs=pl.BlockSpec((RB, D), lambda i: (i, 0)),
    out_shape=jax.ShapeDtypeStruct((NP, D), _F32),
)


def kernel(embedding, down2up_path, same_level_edge_index, up2down_edge_index,
           W1, b1, W2, b2, W3, b3):
    edges = (down2up_path, same_level_edge_index, up2down_edge_index)
    srcs = [e[0].astype(jnp.int32) for e in edges]
    dsts = [e[1].astype(jnp.int32) for e in edges]

    npad = EPAD - E
    fake_src = (jnp.arange(npad, dtype=jnp.int32) * 37) % NP
    fake_dst = N_NODES + jnp.arange(npad, dtype=jnp.int32) % (NP - N_NODES)
    src_conv = []
    for sv in srcs:
        sp = jnp.concatenate([sv, fake_src])
        src_conv.append(jnp.stack([sp, sp + NP]).reshape(
            2 * N_TILES, CONV_NGRP, CONV_GRP, CONV_CH))
    dst_conv = [
        jnp.concatenate([dv, fake_dst]).reshape(
            N_TILES, CONV_NGRP, CONV_GRP, CONV_CH)
        for dv in dsts
    ]

    x = jnp.pad(embedding, ((0, NP - N_NODES), (0, 0)))
    b1r = b1.reshape(1, D)
    b2r = b2.reshape(1, D)
    b3r = b3.reshape(1, D)

    dst_deg = [d.reshape(2 * N_TILES, DEG_NCH, DEG_CH) for d in dsts]
    hm1 = _tc1a(x, W1)
    dp = _deg_call(*dst_deg)
    hs1, base1 = _tc1b(hm1, b1r, dp)
    acc1 = _conv_call(hs1.reshape(2 * NP, HF), src_conv[0], dst_conv[0])
    hs2, base2 = _tcmid01(acc1, base1, dp, W2, b2r)
    acc2 = _conv_call(hs2.reshape(2 * NP, HF), src_conv[1], dst_conv[1])
    hs3, base3 = _tcmid12(acc2, base2, dp, W3, b3r)
    acc3 = _conv_call(hs3.reshape(2 * NP, HF), src_conv[2], dst_conv[2])
    out = _tcf(acc3, base3, dp)
    return out[:N_NODES]
